# Initial kernel scaffold; baseline (speedup 1.0000x reference)
import jax
import jax.numpy as jnp
from jax.experimental import pallas as pl


def kernel(feat, anchor_embeddings, anchor_id, id_table, W_cls, W_o2o, W_reg, W_aux):
    raise NotImplementedError("write your pallas kernel here")


# speedup vs baseline: 3.4093x; 3.4093x over previous
import functools
import math

import jax
import jax.numpy as jnp
import numpy as np
from jax import lax
from jax.experimental import pallas as pl
from jax.experimental.pallas import tpu as pltpu
from jax.experimental.pallas import tpu_sc as plsc

IMG_W = 800.0
IMG_H = 320.0
NUM_OFFSETS = 72
NUM_STRIPS = 71
NUM_FEAT_SAMPLES = 36
NUM_LINE_GROUPS = 4
NMS_THRES = 50.0
CONF_THRES = 0.4
MAX_LANES = 8
PRE_NMS_K = 64
HF, WF = 40, 100

B, N, C = 4, 1000, 64
D = NUM_FEAT_SAMPLES * C

NW = 32
CHUNK = 128
ROWS = B * N * NUM_FEAT_SAMPLES
PER_W = 4608
NCHUNK = PER_W // CHUNK
ROWS_PAD = NW * PER_W


def _sc_gather_make():
    mesh = plsc.VectorSubcoreMesh(core_axis_name="c", subcore_axis_name="s")

    @functools.partial(
        pl.kernel,
        out_type=jax.ShapeDtypeStruct((ROWS_PAD, C), jnp.float32),
        mesh=mesh,
        scratch_types=[
            pltpu.VMEM((NCHUNK, CHUNK), jnp.int32),
            pltpu.VMEM((2, CHUNK, C), jnp.float32),
            pltpu.SemaphoreType.DMA,
        ],
        compiler_params=pltpu.CompilerParams(use_tc_tiling_on_sc=False),
    )
    def k(table_hbm, idx_hbm, out_hbm, idx_v, rows_v, gsem):
        wid = lax.axis_index("s") * 2 + lax.axis_index("c")
        base = wid * PER_W
        pltpu.sync_copy(idx_hbm.at[wid], idx_v)

        def body(j, _):
            slot = lax.rem(j, 2)
            pltpu.async_copy(table_hbm.at[idx_v.at[j]], rows_v.at[slot], gsem).wait()
            pltpu.sync_copy(rows_v.at[slot],
                            out_hbm.at[pl.ds(base + j * CHUNK, CHUNK)])
            return 0

        lax.fori_loop(0, NCHUNK, body, 0)

    return k


_sc_gather = _sc_gather_make()


def _head_mm_kernel(x_ref, w_ref, o_ref):
    o_ref[...] = jnp.dot(x_ref[0], w_ref[...],
                         preferred_element_type=jnp.float32)[None]


def _head_matmul(x_flat, W_all):
    K = W_all.shape[1]
    return pl.pallas_call(
        _head_mm_kernel,
        grid=(B,),
        in_specs=[pl.BlockSpec((1, N, D), lambda b: (b, 0, 0)),
                  pl.BlockSpec((D, K), lambda b: (0, 0))],
        out_specs=pl.BlockSpec((1, N, K), lambda b: (b, 0, 0)),
        out_shape=jax.ShapeDtypeStruct((B, N, K), jnp.float32),
    )(x_flat, W_all)


def _sample_from_anchor(anchor_embeddings):
    ae = jax.lax.stop_gradient(anchor_embeddings)
    theta = ae[..., 0] * math.pi
    rho = ae[..., 1] * IMG_W
    ys = jnp.linspace(0.0, IMG_H - 1.0, NUM_OFFSETS)
    xs = (rho[..., None] - ys * jnp.sin(theta)[..., None]) / (jnp.cos(theta)[..., None] + 1e-6)
    ys_b = jnp.broadcast_to(ys, xs.shape)
    samples_car = jnp.stack([xs, ys_b], axis=-1)
    img_samples = jnp.stack([samples_car[..., 0], IMG_H - 1.0 - samples_car[..., 1]], axis=-1)
    anchor_samples = jnp.flip(samples_car, axis=-2)
    lin = jnp.linspace(0.0, 1.0, NUM_FEAT_SAMPLES)
    si = jnp.flip(NUM_STRIPS - (lin * NUM_STRIPS).astype(jnp.int32), axis=-1)
    grid = img_samples[:, :, si, :]
    grid_norm = grid / jnp.array([IMG_W, IMG_H], dtype=jnp.float32)
    return grid_norm, anchor_samples


def kernel(feat, anchor_embeddings, anchor_id, id_table, W_cls, W_o2o, W_reg, W_aux):
    grid_norm, anchor_samples = _sample_from_anchor(anchor_embeddings)

    px = jnp.clip(jnp.round(grid_norm[..., 0] * (WF - 1)), 0, WF - 1).astype(jnp.int32)
    py = jnp.clip(jnp.round(grid_norm[..., 1] * (HF - 1)), 0, HF - 1).astype(jnp.int32)
    lin_idx = py * WF + px
    gidx = (jnp.arange(B, dtype=jnp.int32)[:, None, None] * (HF * WF) + lin_idx)
    gidx = jnp.pad(gidx.reshape(-1), (0, ROWS_PAD - ROWS)).reshape(NW, NCHUNK, CHUNK)

    flat = feat.transpose(0, 2, 3, 1).reshape(B * HF * WF, C)
    sampled = _sc_gather(flat, gidx)[:ROWS].reshape(B, N, NUM_FEAT_SAMPLES, C)

    id_emb = id_table[anchor_id]
    x_flat = (sampled + id_emb[:, :, None, :]).reshape(B, N, D)

    W_all = jnp.concatenate([W_cls, W_o2o, W_reg, W_aux], axis=1)
    raw = _head_matmul(x_flat, W_all)
    cls_pred = jax.nn.sigmoid(raw[..., 0])
    cls_o2o = jax.nn.sigmoid(raw[..., 1])
    reg_pred = raw[..., 2:76]
    reg_aux = raw[..., 76:84]

    base = jax.lax.stop_gradient(anchor_embeddings)[:, :, None, :]
    line_paras_group_reg = reg_aux.reshape(B, N, NUM_LINE_GROUPS, 2) + base
    end_points = reg_pred[..., 0:2]
    xs_offset = reg_pred[..., 2:]
    lanereg_car_x = anchor_samples[..., 0] + xs_offset * IMG_W
    lanereg_car = jnp.stack([lanereg_car_x, anchor_samples[..., 1]], axis=-1)
    lane_points_img = jnp.stack([lanereg_car[..., 0], IMG_H - 1.0 - lanereg_car[..., 1]], axis=-1)

    def nms_one(scores, lane_xs):
        top_s, top_i = jax.lax.top_k(scores, PRE_NMS_K)
        xk = lane_xs[top_i]
        dist = jnp.mean(jnp.abs(xk[:, None, :] - xk[None, :, :]), axis=-1)
        keep0 = top_s >= CONF_THRES
        idx = jnp.arange(PRE_NMS_K)

        def body(i, keep):
            sup = (dist[i] < NMS_THRES) & (idx > i)
            return jnp.where(keep[i], keep & (~sup), keep)

        keep = jax.lax.fori_loop(0, PRE_NMS_K, body, keep0)
        keep = keep & (jnp.cumsum(keep.astype(jnp.int32)) <= MAX_LANES)
        return top_i, keep

    keep_idx, keep_mask = jax.vmap(nms_one)(cls_pred, lane_points_img[..., 0])
    return (cls_pred, cls_o2o, end_points, xs_offset, line_paras_group_reg,
            lane_points_img, keep_idx, keep_mask)
